# baseline (speedup 1.0000x reference)
import functools

import jax
import jax.numpy as jnp
from jax.experimental import pallas as pl
from jax.experimental.pallas import tpu as pltpu

_B, _I, _E = 16384, 128, 18
_LO, _HI = 64, 71
_NC = _HI - _LO
_W = _I * _E
_CLO = _LO * _E
_CHI = _HI * _E
_TB = 128


def _body(x_ref, m_ref, r2_ref, bd_ref, o_ref):
    x = x_ref[...]
    y = jnp.dot(x, m_ref[...], preferred_element_type=jnp.float32)
    xs_rep = jnp.dot(x, r2_ref[...], preferred_element_type=jnp.float32)
    mi = jax.lax.broadcasted_iota(jnp.int32, xs_rep.shape, 1)
    oh = (xs_rep == (mi % 52).astype(jnp.float32)).astype(jnp.bfloat16)
    y = y + jnp.dot(oh, bd_ref[...], preferred_element_type=jnp.float32)
    o_ref[...] = y.reshape(_TB, _I, _E)


@jax.jit
def kernel(x, card_buffer):
    if x.ndim == 3:
        x = x[:, 0, :]
    B = x.shape[0]
    f32, bf16 = jnp.float32, jnp.bfloat16
    jj = jnp.arange(_W)
    M = ((jj[None, :] // _E == jnp.arange(_I)[:, None])
         & ((jj[None, :] < _CLO) | (jj[None, :] >= _CHI))).astype(bf16)
    R2 = (jnp.arange(_I)[:, None] == _LO + jnp.arange(_NC * 52)[None, :] // 52).astype(bf16)
    BDsmall = (jnp.eye(_NC, dtype=f32)[:, None, :, None]
               * card_buffer[None, :, None, :]).reshape(_NC * 52, _NC * _E)
    BD = jnp.zeros((_NC * 52, _W), f32).at[:, _CLO:_CHI].set(BDsmall).astype(bf16)
    x = x.astype(bf16)

    out = pl.pallas_call(
        _body,
        grid=(B // _TB,),
        in_specs=[
            pl.BlockSpec((_TB, _I), lambda i: (i, 0)),
            pl.BlockSpec((_I, _W), lambda i: (0, 0)),
            pl.BlockSpec((_I, _NC * 52), lambda i: (0, 0)),
            pl.BlockSpec((_NC * 52, _W), lambda i: (0, 0)),
        ],
        out_specs=pl.BlockSpec((_TB, _I, _E), lambda i: (i, 0, 0)),
        out_shape=jax.ShapeDtypeStruct((B, _I, _E), jnp.float32),
        compiler_params=pltpu.CompilerParams(
            dimension_semantics=("parallel",),
        ),
    )(x, M, R2, BD)
    return out


# speedup vs baseline: 2.0916x; 2.0916x over previous
import jax
import jax.numpy as jnp
from jax.experimental import pallas as pl
from jax.experimental.pallas import tpu as pltpu

_B, _I, _E = 16384, 128, 18
_LO, _HI = 64, 71
_NC = _HI - _LO
_TB = 512


def _body(x_ref, xb_ref, r2_ref, bd2_ref, o_ref):
    xs_rep = jnp.dot(xb_ref[...], r2_ref[...], preferred_element_type=jnp.float32)
    mi = jax.lax.broadcasted_iota(jnp.int32, xs_rep.shape, 1)
    oh = (xs_rep == (mi % 52).astype(jnp.float32)).astype(jnp.bfloat16)
    emb = jnp.dot(oh, bd2_ref[...], preferred_element_type=jnp.float32)
    x = x_ref[...]
    o_ref[...] = jnp.broadcast_to(x[None], (_E,) + x.shape)
    for e in range(_E):
        o_ref[e, :, _LO:_HI] = emb[:, e * _NC:(e + 1) * _NC]


@jax.jit
def kernel(x, card_buffer):
    if x.ndim == 3:
        x = x[:, 0, :]
    B = x.shape[0]
    f32, bf16 = jnp.float32, jnp.bfloat16
    R2 = (jnp.arange(_I)[:, None] == _LO + jnp.arange(_NC * 52)[None, :] // 52).astype(bf16)
    kk = jnp.arange(_NC * 52) // 52
    cc = jnp.arange(_NC * 52) % 52
    ee = jnp.arange(_E * _NC) // _NC
    k2 = jnp.arange(_E * _NC) % _NC
    BD2 = (card_buffer[cc[:, None], ee[None, :]]
           * (kk[:, None] == k2[None, :])).astype(bf16)

    out3 = pl.pallas_call(
        _body,
        grid=(B // _TB,),
        in_specs=[
            pl.BlockSpec((_TB, _I), lambda i: (i, 0)),
            pl.BlockSpec((_TB, _I), lambda i: (i, 0)),
            pl.BlockSpec((_I, _NC * 52), lambda i: (0, 0)),
            pl.BlockSpec((_NC * 52, _E * _NC), lambda i: (0, 0)),
        ],
        out_specs=pl.BlockSpec((_E, _TB, _I), lambda i: (0, i, 0)),
        out_shape=jax.ShapeDtypeStruct((_E, B, _I), f32),
        compiler_params=pltpu.CompilerParams(
            dimension_semantics=("arbitrary",),
        ),
    )(x, x.astype(bf16), R2, BD2)
    return jnp.transpose(out3, (1, 2, 0))


# speedup vs baseline: 11.6171x; 5.5541x over previous
import functools

import jax
import jax.numpy as jnp
from jax import lax
from jax.experimental import pallas as pl
from jax.experimental.pallas import tpu as pltpu
from jax.experimental.pallas import tpu_sc as plsc

_B, _I, _E = 16384, 128, 18
_LO, _HI = 64, 71
_NW = 32
_RW = _B // _NW
_HC = _RW // 4


def _sc_body(x_hbm, t_hbm, out_hbm, img0, img1, carr, obak, tvm, sem0, sem1):
    wid = lax.axis_index("s") * 2 + lax.axis_index("c")
    pltpu.sync_copy(t_hbm, tvm)
    lane = lax.iota(jnp.int32, 16)
    lmask = lane < (_HI - _LO)
    imgs = (img0, img1)
    sems = (sem0, sem1)

    for half in range(4):
        row0 = wid * _RW + half * _HC
        pltpu.sync_copy(x_hbm.at[pl.ds(row0, _HC), :], img0)
        pltpu.sync_copy(x_hbm.at[pl.ds(row0, _HC), :], img1)

        def _snap(r, _):
            v = img0[r, pl.ds(_LO, 16)]
            carr[r, :] = v.astype(jnp.int32)
            obak[r, :] = v
            return 0

        lax.fori_loop(0, _HC, _snap, 0)

        pending = [None, None]
        for e in range(_E):
            buf = imgs[e % 2]
            if pending[e % 2] is not None:
                pending[e % 2].wait()

            def _patch(r, _, _eoff=e * 52, _buf=buf):
                idx = carr[r, :] + _eoff
                g = plsc.load_gather(tvm, [idx])
                merged = jnp.where(lmask, g, obak[r, :])
                _buf[r, pl.ds(_LO, 16)] = merged
                return 0

            lax.fori_loop(0, _HC, _patch, 0)
            pending[e % 2] = pltpu.async_copy(
                buf, out_hbm.at[e, pl.ds(row0, _HC), :], sems[e % 2])
        pending[0].wait()
        pending[1].wait()


@jax.jit
def kernel(x, card_buffer):
    if x.ndim == 3:
        x = x[:, 0, :]
    B = x.shape[0]
    f32 = jnp.float32
    T = jnp.concatenate([card_buffer.T.reshape(-1), jnp.zeros((1024 - 52 * _E,), f32)])

    mesh = plsc.VectorSubcoreMesh(core_axis_name="c", subcore_axis_name="s")
    run = functools.partial(
        pl.kernel,
        mesh=mesh,
        compiler_params=pltpu.CompilerParams(needs_layout_passes=False),
        out_type=jax.ShapeDtypeStruct((_E, B, _I), f32),
        scratch_types=[
            pltpu.VMEM((_HC, _I), f32),
            pltpu.VMEM((_HC, _I), f32),
            pltpu.VMEM((_HC, 16), jnp.int32),
            pltpu.VMEM((_HC, 16), f32),
            pltpu.VMEM((1024,), f32),
            pltpu.SemaphoreType.DMA,
            pltpu.SemaphoreType.DMA,
        ],
    )(_sc_body)
    out3 = run(x, T)
    return jnp.transpose(out3, (1, 2, 0))


# speedup vs baseline: 12.2672x; 1.0560x over previous
import functools

import jax
import jax.numpy as jnp
from jax import lax
from jax.experimental import pallas as pl
from jax.experimental.pallas import tpu as pltpu
from jax.experimental.pallas import tpu_sc as plsc

_B, _I, _E = 16384, 128, 18
_LO, _HI = 64, 71
_NW = 32
_RW = _B // _NW
_HC = _RW // 4


def _sc_body(x_hbm, t_hbm, out_hbm, img0, img1, carr, obak, tvm, sem0, sem1):
    wid = lax.axis_index("s") * 2 + lax.axis_index("c")
    pltpu.sync_copy(t_hbm, tvm)
    lane = lax.iota(jnp.int32, 16)
    lmask = lane < (_HI - _LO)
    imgs = (img0, img1)
    sems = (sem0, sem1)

    for half in range(4):
        row0 = wid * _RW + half * _HC
        pltpu.sync_copy(x_hbm.at[pl.ds(row0, _HC), :], img0)

        def _snap(r, _):
            v = img0[r, pl.ds(_LO, 16)]
            carr[r, :] = v.astype(jnp.int32)
            obak[r, :] = v
            for q in (0, 1, 2, 3, 5, 6, 7):
                img1[r, pl.ds(q * 16, 16)] = img0[r, pl.ds(q * 16, 16)]
            return 0

        lax.fori_loop(0, _HC, _snap, 0)

        pending = [None, None]
        for e in range(_E):
            buf = imgs[e % 2]
            if pending[e % 2] is not None:
                pending[e % 2].wait()

            def _patch(r, _, _eoff=e * 52, _buf=buf):
                idx = carr[r, :] + _eoff
                g = plsc.load_gather(tvm, [idx])
                merged = jnp.where(lmask, g, obak[r, :])
                _buf[r, pl.ds(_LO, 16)] = merged
                return 0

            lax.fori_loop(0, _HC, _patch, 0)
            pending[e % 2] = pltpu.async_copy(
                buf, out_hbm.at[e, pl.ds(row0, _HC), :], sems[e % 2])
        pending[0].wait()
        pending[1].wait()


@jax.jit
def kernel(x, card_buffer):
    if x.ndim == 3:
        x = x[:, 0, :]
    B = x.shape[0]
    f32 = jnp.float32
    T = jnp.concatenate([card_buffer.T.reshape(-1), jnp.zeros((1024 - 52 * _E,), f32)])

    mesh = plsc.VectorSubcoreMesh(core_axis_name="c", subcore_axis_name="s")
    run = functools.partial(
        pl.kernel,
        mesh=mesh,
        compiler_params=pltpu.CompilerParams(needs_layout_passes=False),
        out_type=jax.ShapeDtypeStruct((_E, B, _I), f32),
        scratch_types=[
            pltpu.VMEM((_HC, _I), f32),
            pltpu.VMEM((_HC, _I), f32),
            pltpu.VMEM((_HC, 16), jnp.int32),
            pltpu.VMEM((_HC, 16), f32),
            pltpu.VMEM((1024,), f32),
            pltpu.SemaphoreType.DMA,
            pltpu.SemaphoreType.DMA,
        ],
    )(_sc_body)
    out3 = run(x, T)
    return jnp.transpose(out3, (1, 2, 0))


# speedup vs baseline: 12.5888x; 1.0262x over previous
import functools

import jax
import jax.numpy as jnp
from jax import lax
from jax.experimental import pallas as pl
from jax.experimental.pallas import tpu as pltpu
from jax.experimental.pallas import tpu_sc as plsc

_B, _I, _E = 16384, 128, 18
_LO, _HI = 64, 71
_NW = 32
_RW = _B // _NW
_HC = _RW // 4


def _sc_body(x_hbm, t_hbm, out_hbm, img0, img1, xpf, carr, obak, tvm,
             sem0, sem1, sempf):
    wid = lax.axis_index("s") * 2 + lax.axis_index("c")
    pltpu.sync_copy(t_hbm, tvm)
    lane = lax.iota(jnp.int32, 16)
    lmask = lane < (_HI - _LO)
    imgs = (img0, img1)
    sems = (sem0, sem1)

    row_base = wid * _RW
    pltpu.sync_copy(x_hbm.at[pl.ds(row_base, _HC), :], xpf)
    pf = None
    for half in range(4):
        row0 = row_base + half * _HC
        if pf is not None:
            pf.wait()

        def _snap(r, _):
            v = xpf[r, pl.ds(_LO, 16)]
            carr[r, :] = v.astype(jnp.int32)
            obak[r, :] = v
            for q in (0, 1, 2, 3, 5, 6, 7):
                w = xpf[r, pl.ds(q * 16, 16)]
                img0[r, pl.ds(q * 16, 16)] = w
                img1[r, pl.ds(q * 16, 16)] = w
            return 0

        lax.fori_loop(0, _HC, _snap, 0)
        if half < 3:
            pf = pltpu.async_copy(
                x_hbm.at[pl.ds(row0 + _HC, _HC), :], xpf, sempf)

        pending = [None, None]
        for e in range(_E):
            buf = imgs[e % 2]
            if pending[e % 2] is not None:
                pending[e % 2].wait()

            def _patch(r, _, _eoff=e * 52, _buf=buf):
                idx = carr[r, :] + _eoff
                g = plsc.load_gather(tvm, [idx])
                merged = jnp.where(lmask, g, obak[r, :])
                _buf[r, pl.ds(_LO, 16)] = merged
                return 0

            lax.fori_loop(0, _HC, _patch, 0)
            pending[e % 2] = pltpu.async_copy(
                buf, out_hbm.at[e, pl.ds(row0, _HC), :], sems[e % 2])
        pending[0].wait()
        pending[1].wait()


@jax.jit
def kernel(x, card_buffer):
    if x.ndim == 3:
        x = x[:, 0, :]
    B = x.shape[0]
    f32 = jnp.float32
    T = jnp.concatenate([card_buffer.T.reshape(-1), jnp.zeros((1024 - 52 * _E,), f32)])

    mesh = plsc.VectorSubcoreMesh(core_axis_name="c", subcore_axis_name="s")
    run = functools.partial(
        pl.kernel,
        mesh=mesh,
        compiler_params=pltpu.CompilerParams(needs_layout_passes=False),
        out_type=jax.ShapeDtypeStruct((_E, B, _I), f32),
        scratch_types=[
            pltpu.VMEM((_HC, _I), f32),
            pltpu.VMEM((_HC, _I), f32),
            pltpu.VMEM((_HC, _I), f32),
            pltpu.VMEM((_HC, 16), jnp.int32),
            pltpu.VMEM((_HC, 16), f32),
            pltpu.VMEM((1024,), f32),
            pltpu.SemaphoreType.DMA,
            pltpu.SemaphoreType.DMA,
            pltpu.SemaphoreType.DMA,
        ],
    )(_sc_body)
    out3 = run(x, T)
    return jnp.transpose(out3, (1, 2, 0))
